# Initial kernel scaffold; baseline (speedup 1.0000x reference)
import jax
import jax.numpy as jnp
from jax.experimental import pallas as pl


def kernel(client_adj_list, client_node_indices):
    raise NotImplementedError("write your pallas kernel here")


# speedup vs baseline: 1.9918x; 1.9918x over previous
import jax
import jax.numpy as jnp
from jax.experimental import pallas as pl

N_GLOBAL = 2048
TAU_C = 0.5
GAMMA = 1.0
TAU_S = 0.5


def _scatter_vals(local_A, ix):
    gx = ix[:, None]
    gy = ix[None, :]
    return jnp.zeros((N_GLOBAL, N_GLOBAL), dtype=local_A.dtype).at[gx, gy].set(local_A)


def kernel(client_adj_list, client_node_indices):
    GA = jax.vmap(_scatter_vals)(client_adj_list, client_node_indices)
    present = (client_node_indices[:, :, None] == jnp.arange(N_GLOBAL, dtype=client_node_indices.dtype)[None, None, :]).any(axis=1).astype(jnp.float32)
    M = present[:, :, None] * present[:, None, :]

    num_obs = jnp.clip(M.sum(axis=0), 1e-05, None)
    A_mean = (GA * M).sum(axis=0) / num_obs
    is_above = (GA > TAU_C).astype(jnp.float32) * M
    Cv = is_above.sum(axis=0) / num_obs
    sq = (GA - A_mean[None, :, :]) ** 2 * M
    V = sq.sum(axis=0) / num_obs
    S = Cv * jnp.exp(-GAMMA * V)
    M_inv = (S > TAU_S).astype(jnp.float32)
    A_inv = M_inv * A_mean
    A_inv = A_inv * (num_obs > 1e-05).astype(jnp.float32)
    return A_inv


# speedup vs baseline: 1.9953x; 1.0018x over previous
import functools

import jax
import jax.numpy as jnp
from jax import lax
from jax.experimental import pallas as pl
from jax.experimental.pallas import tpu as pltpu
from jax.experimental.pallas import tpu_sc as plsc

N_GLOBAL = 2048
NUM_CLIENTS = 8
L_LOCAL = 1024
TAU_C = 0.5
GAMMA = 1.0
TAU_S = 0.5

ROW_TILE = 256
_LANES = 16


def _scatter_vals(local_A, ix):
    gx = ix[:, None]
    gy = ix[None, :]
    return jnp.zeros((N_GLOBAL, N_GLOBAL), dtype=local_A.dtype).at[gx, gy].set(local_A)


def _presence_kernel(idx_hbm, out_hbm, idx_v, pres_v):
    w = lax.axis_index("s") * 2 + lax.axis_index("c")

    @pl.when(w < NUM_CLIENTS)
    def _():
        pltpu.sync_copy(idx_hbm.at[w], idx_v)

        def zero_body(i, carry):
            pres_v[pl.ds(i * _LANES, _LANES)] = jnp.zeros((_LANES,), jnp.float32)
            return carry

        lax.fori_loop(0, N_GLOBAL // _LANES, zero_body, 0)

        ones = jnp.full((_LANES,), 1.0, jnp.float32)

        def scat_body(i, carry):
            v = idx_v[pl.ds(i * _LANES, _LANES)]
            plsc.store_scatter(pres_v, [v], ones)
            return carry

        lax.fori_loop(0, L_LOCAL // _LANES, scat_body, 0)
        pltpu.sync_copy(pres_v, out_hbm.at[w])


@functools.partial(
    pl.kernel,
    out_type=jax.ShapeDtypeStruct((NUM_CLIENTS, N_GLOBAL), jnp.float32),
    mesh=plsc.VectorSubcoreMesh(core_axis_name="c", subcore_axis_name="s"),
    compiler_params=pltpu.CompilerParams(needs_layout_passes=False),
    scratch_types=[
        pltpu.VMEM((L_LOCAL,), jnp.int32),
        pltpu.VMEM((N_GLOBAL,), jnp.float32),
    ],
)
def _presence(idx_hbm, out_hbm, idx_v, pres_v):
    _presence_kernel(idx_hbm, out_hbm, idx_v, pres_v)


def _stats_kernel(ga_ref, pres_ref, presT_ref, out_ref, n_ref, s_ref, ss_ref, cnt_ref):
    c = pl.program_id(1)

    v = ga_ref[0]
    lane_oh = (lax.broadcasted_iota(jnp.int32, (1, NUM_CLIENTS), 1) == c).astype(jnp.float32)
    sub_oh = (lax.broadcasted_iota(jnp.int32, (NUM_CLIENTS, 1), 0) == c).astype(jnp.float32)
    rowp = jnp.sum(presT_ref[...] * lane_oh, axis=1, keepdims=True)
    colp = jnp.sum(pres_ref[...] * sub_oh, axis=0, keepdims=True)
    m = rowp * colp
    sv = v * m
    above = jnp.where(v > TAU_C, m, 0.0)

    @pl.when(c == 0)
    def _():
        n_ref[...] = m
        s_ref[...] = sv
        ss_ref[...] = sv * v
        cnt_ref[...] = above

    @pl.when(c > 0)
    def _():
        n_ref[...] += m
        s_ref[...] += sv
        ss_ref[...] += sv * v
        cnt_ref[...] += above

    @pl.when(c == NUM_CLIENTS - 1)
    def _():
        n = n_ref[...]
        nc = jnp.maximum(n, 1e-05)
        mean = s_ref[...] / nc
        var = ss_ref[...] / nc - mean * mean
        cfrac = cnt_ref[...] / nc
        score = cfrac * jnp.exp(-GAMMA * var)
        keep = jnp.logical_and(score > TAU_S, n > 1e-05)
        out_ref[...] = jnp.where(keep, mean, 0.0)


def _fused_stats(GA, pres, presT):
    grid = (N_GLOBAL // ROW_TILE, NUM_CLIENTS)
    return pl.pallas_call(
        _stats_kernel,
        grid=grid,
        in_specs=[
            pl.BlockSpec((1, ROW_TILE, N_GLOBAL), lambda t, c: (c, t, 0)),
            pl.BlockSpec((NUM_CLIENTS, N_GLOBAL), lambda t, c: (0, 0)),
            pl.BlockSpec((ROW_TILE, NUM_CLIENTS), lambda t, c: (t, 0)),
        ],
        out_specs=pl.BlockSpec((ROW_TILE, N_GLOBAL), lambda t, c: (t, 0)),
        out_shape=jax.ShapeDtypeStruct((N_GLOBAL, N_GLOBAL), jnp.float32),
        scratch_shapes=[pltpu.VMEM((ROW_TILE, N_GLOBAL), jnp.float32)] * 4,
        compiler_params=pltpu.CompilerParams(
            dimension_semantics=("parallel", "arbitrary"),
        ),
    )(GA, pres, presT)


def kernel(client_adj_list, client_node_indices):
    GA = jax.vmap(_scatter_vals)(client_adj_list, client_node_indices)
    pres = _presence(client_node_indices.astype(jnp.int32))
    presT = pres.T
    return _fused_stats(GA, pres, presT)
